# Initial kernel scaffold; baseline (speedup 1.0000x reference)
import jax
import jax.numpy as jnp
from jax.experimental import pallas as pl


def kernel(patch_x, patch_embedding, affinity, noise_eps, expert_W, expert_b):
    raise NotImplementedError("write your pallas kernel here")


# speedup vs baseline: 1.2658x; 1.2658x over previous
import functools

import jax
import jax.numpy as jnp
from jax.experimental import pallas as pl
from jax.experimental.pallas import tpu as pltpu

TOP_K = 2
N_EXPERTS = 8
D_MODEL = 768
N_TOK = 4096
P = 2

ROWS = N_TOK * P
BLK_ROWS = 1024


def _moe_dense_body(aff_ref, eps_ref, x_ref, w_ref, b_ref, out_ref):
    aff = aff_ref[...]
    eps = eps_ref[...]
    sp = jnp.maximum(aff, 0.0) + jnp.log1p(jnp.exp(-jnp.abs(aff)))
    noisy = aff + eps * sp

    lane = jax.lax.broadcasted_iota(jnp.int32, noisy.shape, 1)
    cnt = jnp.zeros(noisy.shape, jnp.int32)
    for j in range(N_EXPERTS):
        vj = noisy[:, j:j + 1]
        gt = vj > noisy
        tie = jnp.logical_and(vj == noisy, j < lane)
        cnt = cnt + jnp.logical_or(gt, tie).astype(jnp.int32)
    sel = cnt < TOP_K

    m1 = jnp.max(noisy, axis=1, keepdims=True)
    e = jnp.where(sel, jnp.exp(noisy - m1), 0.0)
    g = e / jnp.sum(e, axis=1, keepdims=True)

    x = x_ref[...]
    acc = jnp.zeros((x.shape[0], D_MODEL), jnp.float32)
    for i in range(N_EXPERTS):
        y = jnp.dot(x, w_ref[i], preferred_element_type=jnp.float32)
        y = y + b_ref[i][None, :]
        acc = acc + y * g[:, i:i + 1]
    out_ref[...] = acc


@jax.jit
def _moe_dense(aff_rows, eps_rows, x, expert_W, expert_b):
    grid = (ROWS // BLK_ROWS,)
    return pl.pallas_call(
        _moe_dense_body,
        grid=grid,
        in_specs=[
            pl.BlockSpec((BLK_ROWS, N_EXPERTS), lambda b: (b, 0)),
            pl.BlockSpec((BLK_ROWS, N_EXPERTS), lambda b: (b, 0)),
            pl.BlockSpec((BLK_ROWS, D_MODEL), lambda b: (b, 0)),
            pl.BlockSpec((N_EXPERTS, D_MODEL, D_MODEL), lambda b: (0, 0, 0)),
            pl.BlockSpec((N_EXPERTS, D_MODEL), lambda b: (0, 0)),
        ],
        out_specs=pl.BlockSpec((BLK_ROWS, D_MODEL), lambda b: (b, 0)),
        out_shape=jax.ShapeDtypeStruct((ROWS, D_MODEL), jnp.float32),
    )(aff_rows, eps_rows, x, expert_W, expert_b)


def kernel(patch_x, patch_embedding, affinity, noise_eps, expert_W, expert_b):
    x = patch_x.reshape(ROWS, D_MODEL)
    aff_rows = jnp.repeat(affinity, P, axis=0)
    eps_rows = jnp.repeat(noise_eps, P, axis=0)
    out = _moe_dense(aff_rows, eps_rows, x, expert_W, expert_b)
    return out.reshape(N_TOK, P, D_MODEL)
